# Initial kernel scaffold; baseline (speedup 1.0000x reference)
import jax
import jax.numpy as jnp
from jax.experimental import pallas as pl


def kernel(encoder_hidden, E, W_ih, W_hh, b_ih, b_hh, W_out, b_out):
    raise NotImplementedError("write your pallas kernel here")


# speedup vs baseline: 4.0135x; 4.0135x over previous
import jax
import jax.numpy as jnp
from jax.experimental import pallas as pl
from jax.experimental.pallas import tpu as pltpu

_B, _H, _V, _K, _T = 32, 1024, 1000, 8, 16
_BK = _B * _K
_NEG = float("-inf")


def _fwd_kernel(h0_ref, E_ref, Wih_ref, Whh_ref, bih_ref, bhh_ref,
                Wout_ref, bout_ref,
                logp_hbm, sym_ref, pred_ref, final_ref,
                logp_vmem, dma_sem):
    f32 = jnp.float32
    h = h0_ref[...]
    E = E_ref[...]
    Wih = Wih_ref[...]
    Whh = Whh_ref[...]
    Wout = Wout_ref[...]
    bih = bih_ref[...]
    bhh = bhh_ref[...]
    bout = bout_ref[...]

    row = jax.lax.broadcasted_iota(jnp.int32, (_BK, 1), 0)
    seq0 = jnp.where(row % _K == 0, 0.0, _NEG).astype(f32)
    inp0 = jnp.zeros((_BK, 1), jnp.int32)

    vocab_iota = jax.lax.broadcasted_iota(jnp.int32, (_BK, _V), 1)
    col_bk = jax.lax.broadcasted_iota(jnp.int32, (_BK, _BK), 1)
    col_t = jax.lax.broadcasted_iota(jnp.int32, (_BK, _T), 1)
    flat_iota = (_V * jax.lax.broadcasted_iota(jnp.int32, (_B, _K, _V), 1)
                 + jax.lax.broadcasted_iota(jnp.int32, (_B, _K, _V), 2))
    beam_base = _K * jax.lax.broadcasted_iota(jnp.int32, (_B, _K, 1), 0)

    def step(t, carry):
        h, seq, inp, sym_mat, pred_mat, _ = carry
        x = jnp.dot((inp == vocab_iota).astype(jnp.bfloat16), E,
                    preferred_element_type=f32)
        gi = jnp.dot(x.astype(jnp.bfloat16), Wih,
                     preferred_element_type=f32) + bih
        gh = jnp.dot(h.astype(jnp.bfloat16), Whh,
                     preferred_element_type=f32) + bhh
        r = jax.nn.sigmoid(gi[:, :_H] + gh[:, :_H])
        z = jax.nn.sigmoid(gi[:, _H:2 * _H] + gh[:, _H:2 * _H])
        n = jnp.tanh(gi[:, 2 * _H:] + r * gh[:, 2 * _H:])
        h = (1.0 - z) * n + z * h

        logits = jnp.dot(h.astype(jnp.bfloat16), Wout,
                         preferred_element_type=f32) + bout
        mx = jnp.max(logits, axis=-1, keepdims=True)
        sh = logits - mx
        logp = sh - jnp.log(jnp.sum(jnp.exp(sh), axis=-1, keepdims=True))
        logp_vmem[...] = logp
        cp = pltpu.make_async_copy(logp_vmem, logp_hbm.at[t], dma_sem)
        cp.start()
        cp.wait()

        ss = (seq + logp).reshape(_B, _K, _V)
        taken = jnp.zeros((_B, _K, _V), jnp.bool_)
        cand_cols, score_cols = [], []
        for _ in range(_K):
            wk = jnp.where(taken, _NEG, ss)
            m = jnp.max(wk, axis=(1, 2), keepdims=True)
            elig = (wk == m) & ~taken
            sel = jnp.min(jnp.where(elig, flat_iota, _K * _V),
                          axis=(1, 2), keepdims=True)
            taken = taken | (flat_iota == sel)
            cand_cols.append(sel)
            score_cols.append(m)
        cand = jnp.concatenate(cand_cols, axis=1)
        scores = jnp.concatenate(score_cols, axis=1)

        inp = (cand % _V).reshape(_BK, 1)
        pred = (cand // _V + beam_base).reshape(_BK, 1)
        seq = scores.reshape(_BK, 1)

        perm = (pred == col_bk).astype(f32)
        h = jnp.dot(perm, h, preferred_element_type=f32,
                    precision=jax.lax.Precision.HIGHEST)

        sym_mat = jnp.where(col_t == t, inp.astype(f32), sym_mat)
        pred_mat = jnp.where(col_t == t, pred.astype(f32), pred_mat)
        seq = jnp.where(inp == 1, _NEG, seq)
        return h, seq, inp, sym_mat, pred_mat, scores

    zeros_t = jnp.zeros((_BK, _T), f32)
    carry = (h, seq0, inp0, zeros_t, zeros_t,
             jnp.zeros((_B, _K, 1), f32))
    _, _, _, sym_mat, pred_mat, scores = jax.lax.fori_loop(
        0, _T, step, carry)
    sym_ref[...] = sym_mat
    pred_ref[...] = pred_mat
    final_ref[...] = scores


def _bwd_kernel(logp_ref, sym_ref, pred_ref, final_ref,
                dec_ref, seqsym_ref, score_ref):
    f32 = jnp.float32
    final = final_ref[...]
    kiota = jax.lax.broadcasted_iota(jnp.int32, (_B, _K, 1), 1)
    taken = jnp.zeros((_B, _K, 1), jnp.bool_)
    order_cols, val_cols = [], []
    for _ in range(_K):
        wk = jnp.where(taken, _NEG, final)
        m = jnp.max(wk, axis=(1, 2), keepdims=True)
        elig = (wk == m) & ~taken
        sel = jnp.min(jnp.where(elig, kiota, _K),
                      axis=(1, 2), keepdims=True)
        taken = taken | (kiota == sel)
        order_cols.append(sel)
        val_cols.append(m)
    order = jnp.concatenate(order_cols, axis=1)
    score_ref[...] = jnp.concatenate(val_cols, axis=1)

    row = jax.lax.broadcasted_iota(jnp.int32, (_BK, 1), 0)
    ptr = order.reshape(_BK, 1) + (row // _K) * _K
    col_bk = jax.lax.broadcasted_iota(jnp.int32, (_BK, _BK), 1)
    col_b = jax.lax.broadcasted_iota(jnp.int32, (_B, _BK), 1)
    sel0 = (col_b == _K * jax.lax.broadcasted_iota(jnp.int32, (_B, _BK), 0)
            ).astype(f32)
    for t in range(_T - 1, -1, -1):
        perm = (ptr == col_bk).astype(f32)
        syms = jnp.dot(perm, sym_ref[:, t:t + 1],
                       preferred_element_type=f32,
                    precision=jax.lax.Precision.HIGHEST)
        seqsym_ref[t] = jnp.floor(syms + 0.5).astype(jnp.int32)

        ptr0 = jnp.dot(sel0, ptr.astype(f32),
                       preferred_element_type=f32,
                    precision=jax.lax.Precision.HIGHEST)
        ptr0 = jnp.floor(ptr0 + 0.5).astype(jnp.int32)
        oh0 = (ptr0 == col_b).astype(f32)
        dec_ref[t] = jnp.dot(oh0, logp_ref[t], preferred_element_type=f32,
                    precision=jax.lax.Precision.HIGHEST)

        ptrf = jnp.dot(perm, pred_ref[:, t:t + 1],
                       preferred_element_type=f32,
                    precision=jax.lax.Precision.HIGHEST)
        ptr = jnp.floor(ptrf + 0.5).astype(jnp.int32)


def kernel(encoder_hidden, E, W_ih, W_hh, b_ih, b_hh, W_out, b_out):
    f32 = jnp.float32
    h0 = jnp.tile(encoder_hidden[0], (_K, 1)).astype(f32)

    logp, sym, pred, final = pl.pallas_call(
        _fwd_kernel,
        out_shape=[
            jax.ShapeDtypeStruct((_T, _BK, _V), f32),
            jax.ShapeDtypeStruct((_BK, _T), f32),
            jax.ShapeDtypeStruct((_BK, _T), f32),
            jax.ShapeDtypeStruct((_B, _K, 1), f32),
        ],
        out_specs=[
            pl.BlockSpec(memory_space=pl.ANY),
            pl.BlockSpec(memory_space=pltpu.MemorySpace.VMEM),
            pl.BlockSpec(memory_space=pltpu.MemorySpace.VMEM),
            pl.BlockSpec(memory_space=pltpu.MemorySpace.VMEM),
        ],
        scratch_shapes=[
            pltpu.VMEM((_BK, _V), f32),
            pltpu.SemaphoreType.DMA,
        ],
    )(h0, E.astype(jnp.bfloat16), W_ih.T.astype(jnp.bfloat16),
      W_hh.T.astype(jnp.bfloat16),
      b_ih[None].astype(f32), b_hh[None].astype(f32),
      W_out.T.astype(jnp.bfloat16), b_out[None].astype(f32))

    dec_out, seqsym, score = pl.pallas_call(
        _bwd_kernel,
        out_shape=[
            jax.ShapeDtypeStruct((_T, _B, _V), f32),
            jax.ShapeDtypeStruct((_T, _BK, 1), jnp.int32),
            jax.ShapeDtypeStruct((_B, _K, 1), f32),
        ],
    )(logp, sym, pred, final)

    topk_seq = seqsym.reshape(_T, _B, _K)
    sorted_scores = score.reshape(_B, _K)
    return dec_out, topk_seq, sorted_scores
